# baseline (speedup 1.0000x reference)
import functools

import jax
import jax.numpy as jnp
from jax import lax
from jax.experimental import pallas as pl
from jax.experimental.pallas import tpu as pltpu
from jax.experimental.pallas import tpu_sc as plsc

_BATCH = 64
_BEAM = 4
_VOCAB = 100000
_PAD = 1
_EOS = 2

_ROWS = _BATCH * _BEAM
_NC, _NS, _L = 2, 16, 16
_NW = _NC * _NS
_ROWS_PER_W = _ROWS // _NW
_NV = _VOCAB // _L
_CAP = 128
_NEG = -1e30


def _sc_topk_candidates(lprobs):
    mesh = plsc.VectorSubcoreMesh(core_axis_name="c", subcore_axis_name="s")

    @functools.partial(
        pl.kernel,
        out_type=(
            jax.ShapeDtypeStruct((_ROWS, _CAP), jnp.float32),
            jax.ShapeDtypeStruct((_ROWS, _CAP), jnp.int32),
        ),
        mesh=mesh,
        compiler_params=pltpu.CompilerParams(needs_layout_passes=False),
        scratch_types=[
            pltpu.VMEM((_VOCAB,), jnp.float32),
            pltpu.VMEM((_CAP,), jnp.float32),
            pltpu.VMEM((_CAP,), jnp.int32),
            pltpu.SMEM((1,), jnp.int32),
        ],
    )
    def k(lprobs_hbm, cand_v_hbm, cand_i_hbm, buf, cv, ci, off):
        wid = lax.axis_index("s") * _NC + lax.axis_index("c")
        lanes = lax.iota(jnp.int32, _L)

        for r in range(_ROWS_PER_W):
            row = wid * _ROWS_PER_W + r
            pltpu.sync_copy(lprobs_hbm.at[row], buf)
            x0 = buf[pl.ds(0, _L)]
            forbidden = (lanes == _PAD) | (lanes == _EOS)
            buf[pl.ds(0, _L)] = jnp.where(forbidden,
                                          jnp.full((_L,), _NEG, jnp.float32),
                                          x0)

            def p1(j, acc):
                return jnp.maximum(acc, buf[pl.ds(j * _L, _L)])
            acc = lax.fori_loop(0, _NV, p1, jnp.full((_L,), _NEG, jnp.float32))
            rem = acc
            thr = jnp.float32(0)
            for _ in range(8):
                thr = jnp.max(rem, axis=0)
                rem = jnp.where(rem >= jnp.full((_L,), thr), _NEG, rem)
            tvec = jnp.full((_L,), thr)

            for c in range(_CAP // _L):
                cv[pl.ds(c * _L, _L)] = jnp.full((_L,), _NEG, jnp.float32)
                ci[pl.ds(c * _L, _L)] = jnp.zeros((_L,), jnp.int32)
            off[0] = 0

            def p2(j, carry):
                x = buf[pl.ds(j * _L, _L)]
                msk = x >= tvec

                @pl.when(jnp.any(msk))
                def _():
                    o = jnp.minimum(off[0], _CAP - _L)
                    plsc.store_compressed(cv.at[pl.ds(o, _L)], x, mask=msk)
                    plsc.store_compressed(ci.at[pl.ds(o, _L)],
                                          j * _L + lanes, mask=msk)
                    off[0] = o + jnp.sum(msk.astype(jnp.int32))
                return carry
            lax.fori_loop(0, _NV, p2, jnp.int32(0))

            pltpu.sync_copy(cv, cand_v_hbm.at[row])
            pltpu.sync_copy(ci, cand_i_hbm.at[row])

    return k(lprobs)


def _tc_lse(lprobs):
    def body(x_ref, o_ref):
        x = x_ref[...]
        m = jnp.max(x, axis=1, keepdims=True)
        s = jnp.sum(jnp.exp(x - m), axis=1, keepdims=True)
        o_ref[...] = m + jnp.log(s)

    return pl.pallas_call(
        body,
        grid=(_ROWS // 8,),
        in_specs=[pl.BlockSpec((8, _VOCAB), lambda i: (i, 0))],
        out_specs=pl.BlockSpec((8, 1), lambda i: (i, 0)),
        out_shape=jax.ShapeDtypeStruct((_ROWS, 1), jnp.float32),
    )(lprobs)


def _tc_merge(cand_v, cand_i, lse):
    w = _BEAM * _CAP

    def body(v_ref, i_ref, l_ref, s_ref, t_ref, b_ref):
        lse4 = l_ref[...]
        adj = jnp.reshape(
            jnp.broadcast_to(lse4[:, :, None], (_BATCH, _BEAM, _CAP)),
            (_BATCH, w))
        scores = v_ref[...] - adj
        toks = i_ref[...]
        col = lax.broadcasted_iota(jnp.int32, (_BATCH, w), 1)
        big = jnp.int32(1 << 30)
        for kk in range(2 * _BEAM):
            m = jnp.max(scores, axis=1, keepdims=True)
            pos = jnp.min(jnp.where(scores == m, col, big),
                          axis=1, keepdims=True)
            sel = col == pos
            s_ref[:, kk] = m[:, 0]
            t_ref[:, kk] = jnp.sum(jnp.where(sel, toks, 0), axis=1)
            b_ref[:, kk] = pos[:, 0] // _CAP
            scores = jnp.where(sel, jnp.float32(-3e38), scores)

    return pl.pallas_call(
        body,
        out_shape=(
            jax.ShapeDtypeStruct((_BATCH, 2 * _BEAM), jnp.float32),
            jax.ShapeDtypeStruct((_BATCH, 2 * _BEAM), jnp.int32),
            jax.ShapeDtypeStruct((_BATCH, 2 * _BEAM), jnp.int32),
        ),
    )(cand_v.reshape(_BATCH, w), cand_i.reshape(_BATCH, w),
      lse.reshape(_BATCH, _BEAM))


def kernel(lprobs):
    cand_v, cand_i = _sc_topk_candidates(lprobs)
    lse = _tc_lse(lprobs)
    return _tc_merge(cand_v, cand_i, lse)


# speedup vs baseline: 3.2604x; 3.2604x over previous
import functools

import jax
import jax.numpy as jnp
from jax import lax
from jax.experimental import pallas as pl
from jax.experimental.pallas import tpu as pltpu
from jax.experimental.pallas import tpu_sc as plsc

_BATCH = 64
_BEAM = 4
_VOCAB = 100000
_PAD = 1
_EOS = 2

_ROWS = _BATCH * _BEAM
_NC, _NS, _L = 2, 16, 16
_NW = _NC * _NS
_RPW = _ROWS // _NW
_NV = _VOCAB // _L
_BLK = 32
_NBLK = (_NV + _BLK - 1) // _BLK
_NVP = _NBLK * _BLK
_CHUNK = 20000
_NCHUNK = _VOCAB // _CHUNK
_CVREG = _CHUNK // _L
_P1U = 10
_CAP = 128
_NEG = -1e30


def _sc_scan(flat):
    mesh = plsc.VectorSubcoreMesh(core_axis_name="c", subcore_axis_name="s")

    @functools.partial(
        pl.kernel,
        out_type=(
            jax.ShapeDtypeStruct((_ROWS * _CAP,), jnp.float32),
            jax.ShapeDtypeStruct((_ROWS * _CAP,), jnp.int32),
            jax.ShapeDtypeStruct((_ROWS * _L,), jnp.float32),
        ),
        mesh=mesh,
        compiler_params=pltpu.CompilerParams(needs_layout_passes=False),
        scratch_types=[
            pltpu.VMEM((_NVP * _L,), jnp.float32),
            pltpu.VMEM((_RPW * _CAP,), jnp.float32),
            pltpu.VMEM((_RPW * _CAP,), jnp.int32),
            pltpu.VMEM((_RPW * _L,), jnp.float32),
            pltpu.VMEM((_L,), jnp.int32),
            pltpu.SemaphoreType.DMA,
            pltpu.SemaphoreType.DMA,
            pltpu.SemaphoreType.DMA,
            pltpu.SemaphoreType.DMA,
            pltpu.SemaphoreType.DMA,
        ],
    )
    def k(x_hbm, cv_hbm, ci_hbm, st_hbm, buf, cvs, cis, sts, offv, *sems):
        wid = lax.axis_index("s") * _NC + lax.axis_index("c")
        lanes = lax.iota(jnp.int32, _L)
        negs = jnp.full((_L,), _NEG, jnp.float32)

        for u in range(_NV, _NVP):
            buf[pl.ds(u * _L, _L)] = negs

        def row_body(r, _):
            base = (wid * _RPW + r) * _VOCAB
            cp = [
                pltpu.async_copy(
                    x_hbm.at[pl.ds(base + c * _CHUNK, _CHUNK)],
                    buf.at[pl.ds(c * _CHUNK, _CHUNK)],
                    sems[c],
                )
                for c in range(_NCHUNK)
            ]

            acc = negs
            for c in range(_NCHUNK):
                cp[c].wait()
                if c == 0:
                    x0 = buf[pl.ds(0, _L)]
                    forbidden = (lanes == _PAD) | (lanes == _EOS)
                    buf[pl.ds(0, _L)] = jnp.where(forbidden, negs, x0)

                def p1(i, a, c=c):
                    b0 = (c * _CVREG + i * _P1U) * _L
                    for u in range(_P1U):
                        a = jnp.maximum(a, buf[pl.ds(b0 + u * _L, _L)])
                    return a
                acc = lax.fori_loop(0, _CVREG // _P1U, p1, acc)

            rem = acc
            m = jnp.max(acc, axis=0)
            thr = m
            for _i in range(8):
                thr = jnp.max(rem, axis=0)
                rem = jnp.where(rem >= jnp.full((_L,), thr), negs, rem)
            tvec = jnp.full((_L,), thr)
            mvec = jnp.full((_L,), m)

            for u in range(_CAP // _L):
                cvs[pl.ds(r * _CAP + u * _L, _L)] = negs
                cis[pl.ds(r * _CAP + u * _L, _L)] = jnp.zeros((_L,), jnp.int32)
            offv[...] = jnp.full((_L,), r * _CAP, jnp.int32)
            lim = jnp.full((_L,), r * _CAP + _CAP - 1, jnp.int32)

            def blk(b, sa):
                sa0, sa1 = sa
                mor0 = None
                mor1 = None
                for u in range(_BLK):
                    x = buf[pl.ds((b * _BLK + u) * _L, _L)]
                    e = jnp.exp(x - mvec)
                    msk = x >= tvec
                    if u % 2 == 0:
                        sa0 = sa0 + e
                        mor0 = msk if mor0 is None else (mor0 | msk)
                    else:
                        sa1 = sa1 + e
                        mor1 = msk if mor1 is None else (mor1 | msk)

                @pl.when(jnp.any(mor0 | mor1))
                def _():
                    off = offv[...]
                    for u in range(_BLK):
                        x = buf[pl.ds((b * _BLK + u) * _L, _L)]
                        msk = x >= tvec
                        ones = jnp.where(msk, 1, 0).astype(jnp.int32)
                        pc = plsc.cumsum(ones)
                        cnt = plsc.all_reduce_population_count(msk)
                        idx = jnp.minimum(off + pc - 1, lim)
                        tok = (b * _BLK + u) * _L + lanes
                        plsc.store_scatter(cvs, [idx], x, mask=msk)
                        plsc.store_scatter(cis, [idx], tok, mask=msk)
                        off = off + cnt
                    offv[...] = off
                return sa0, sa1

            z = jnp.zeros((_L,), jnp.float32)
            sa0, sa1 = lax.fori_loop(0, _NBLK, blk, (z, z))
            s = jnp.sum(sa0 + sa1, axis=0)
            sts[pl.ds(r * _L, _L)] = jnp.where(
                lanes == 0, jnp.full((_L,), m),
                jnp.where(lanes == 1, jnp.full((_L,), s), z))
            return 0

        lax.fori_loop(0, _RPW, row_body, 0)
        pltpu.sync_copy(cvs, cv_hbm.at[pl.ds(wid * _RPW * _CAP, _RPW * _CAP)])
        pltpu.sync_copy(cis, ci_hbm.at[pl.ds(wid * _RPW * _CAP, _RPW * _CAP)])
        pltpu.sync_copy(sts, st_hbm.at[pl.ds(wid * _RPW * _L, _RPW * _L)])

    return k(flat)


def _tc_merge(cand_v, cand_i, m4, s4):
    w = _BEAM * _CAP

    def body(v_ref, i_ref, m_ref, s_ref, os_ref, ot_ref, ob_ref):
        lse4 = m_ref[...] + jnp.log(s_ref[...])
        adj = jnp.reshape(
            jnp.broadcast_to(lse4[:, :, None], (_BATCH, _BEAM, _CAP)),
            (_BATCH, w))
        scores = v_ref[...] - adj
        toks = i_ref[...]
        col = lax.broadcasted_iota(jnp.int32, (_BATCH, w), 1)
        big = jnp.int32(1 << 30)
        for kk in range(2 * _BEAM):
            mx = jnp.max(scores, axis=1, keepdims=True)
            pos = jnp.min(jnp.where(scores == mx, col, big),
                          axis=1, keepdims=True)
            sel = col == pos
            os_ref[:, kk] = mx[:, 0]
            ot_ref[:, kk] = jnp.sum(jnp.where(sel, toks, 0), axis=1)
            ob_ref[:, kk] = pos[:, 0] // _CAP
            scores = jnp.where(sel, jnp.float32(-3e38), scores)

    return pl.pallas_call(
        body,
        out_shape=(
            jax.ShapeDtypeStruct((_BATCH, 2 * _BEAM), jnp.float32),
            jax.ShapeDtypeStruct((_BATCH, 2 * _BEAM), jnp.int32),
            jax.ShapeDtypeStruct((_BATCH, 2 * _BEAM), jnp.int32),
        ),
    )(cand_v.reshape(_BATCH, w), cand_i.reshape(_BATCH, w), m4, s4)


def kernel(lprobs):
    cand_v, cand_i, stats = _sc_scan(lprobs.reshape(-1))
    st = stats.reshape(_ROWS, _L)
    m4 = st[:, 0].reshape(_BATCH, _BEAM)
    s4 = st[:, 1].reshape(_BATCH, _BEAM)
    return _tc_merge(cand_v, cand_i, m4, s4)
